# Initial kernel scaffold; baseline (speedup 1.0000x reference)
import jax
import jax.numpy as jnp
from jax.experimental import pallas as pl


def kernel(x, edge_index, W1, b1, W2, b2, Wg_rel, Wg_root, bg):
    raise NotImplementedError("write your pallas kernel here")


# speedup vs baseline: 1.2000x; 1.2000x over previous
import math

import functools

import jax
import jax.numpy as jnp
from jax import lax
from jax.experimental import pallas as pl
from jax.experimental.pallas import tpu as pltpu
from jax.experimental.pallas import tpu_sc as plsc

N = 10000
E = 320000
D_IN = 128
D_H = 256
K = int(math.ceil(0.5 * N))


def _identity_body(x_ref, o_ref):
    o_ref[...] = x_ref[...]


def _pallas_identity(x):
    return pl.pallas_call(
        _identity_body,
        out_shape=jax.ShapeDtypeStruct(x.shape, x.dtype),
    )(x)


def _mm_body(a_ref, b_ref, o_ref):
    o_ref[...] = jnp.dot(a_ref[...], b_ref[...])


def _pallas_mm(a, b, bm=1000):
    m, k = a.shape
    n = b.shape[1]
    return pl.pallas_call(
        _mm_body,
        grid=(m // bm,),
        in_specs=[
            pl.BlockSpec((bm, k), lambda i: (i, 0)),
            pl.BlockSpec((k, n), lambda i: (0, 0)),
        ],
        out_specs=pl.BlockSpec((bm, n), lambda i: (i, 0)),
        out_shape=jax.ShapeDtypeStruct((m, n), jnp.float32),
    )(a, b)


def _score_body(agg_ref, h_ref, wr_ref, wo_ref, bg_ref, o_ref):
    z = jnp.dot(agg_ref[...], wr_ref[...]) + jnp.dot(h_ref[...], wo_ref[...]) + bg_ref[0, 0]
    o_ref[...] = jnp.tanh(z)


def _pallas_score(agg, h, Wrel, Wroot, bg, bm=1000):
    m, k = agg.shape
    return pl.pallas_call(
        _score_body,
        grid=(m // bm,),
        in_specs=[
            pl.BlockSpec((bm, k), lambda i: (i, 0)),
            pl.BlockSpec((bm, k), lambda i: (i, 0)),
            pl.BlockSpec((k, 1), lambda i: (0, 0)),
            pl.BlockSpec((k, 1), lambda i: (0, 0)),
            pl.BlockSpec((1, 1), lambda i: (0, 0)),
        ],
        out_specs=pl.BlockSpec((bm, 1), lambda i: (i, 0)),
        out_shape=jax.ShapeDtypeStruct((m, 1), jnp.float32),
    )(agg, h, Wrel, Wroot, bg.reshape(1, 1))


_CAP = 160


def _csr(dst, n):
    order = jnp.argsort(dst, stable=True)
    indeg = jnp.zeros((n,), jnp.int32).at[dst].add(1)
    base = jnp.cumsum(indeg) - indeg
    return order, indeg, base


def _ordered_scatter_rows(vals_sorted, base, indeg, n, d_feat):
    acc = jnp.zeros((n, d_feat), jnp.float32)
    e = vals_sorted.shape[0]

    def body(k, acc):
        pos = jnp.minimum(base + k, e - 1)
        valid = k < indeg
        v = jnp.where(valid[:, None], vals_sorted[pos], 0.0)
        return acc + v

    return jax.lax.fori_loop(0, _CAP, body, acc)


_NW = 32
_SLAB = 320
_NPAD = _NW * _SLAB
_CH = 2000
_NCHUNK = E // _CH
_ROWL = E + 16 * _NCHUNK
_TRASH = _SLAB

_sc_mesh = plsc.VectorSubcoreMesh(core_axis_name="c", subcore_axis_name="s")


def _scalar(v):
    return jnp.max(v) if getattr(v, "ndim", 0) else v


def _prep_body(dst_hbm, src_hbm, indeg_hbm, cnt_hbm, srcc_hbm, drelc_hbm,
               dbuf, sbuf, stag_s, stag_d, hist, cbuf):
    w = lax.axis_index("c") * 16 + lax.axis_index("s")
    d0 = w * _SLAB
    zero16 = jnp.zeros((16,), jnp.int32)
    for g in range(_SLAB // 16):
        hist[pl.ds(g * 16, 16)] = zero16
    ones16 = jnp.ones((16,), jnp.int32)
    sent16 = jnp.full((16,), _TRASH, jnp.int32)

    def chunk_body(c, cnt16):
        pltpu.sync_copy(dst_hbm.at[pl.ds(pl.multiple_of(c * _CH, 8), _CH)], dbuf)
        pltpu.sync_copy(src_hbm.at[pl.ds(pl.multiple_of(c * _CH, 8), _CH)], sbuf)

        def grp(g, nc):
            dv = dbuf[pl.ds(g * 16, 16)]
            sv = sbuf[pl.ds(g * 16, 16)]
            rel = dv - d0
            m = (rel >= 0) & (rel < _SLAB)
            relc = jnp.where(m, rel, 0)
            plsc.addupdate_scatter(hist, [relc], ones16, mask=m)
            plsc.store_compressed(stag_s.at[pl.ds(nc, 16)], sv, mask=m)
            plsc.store_compressed(stag_d.at[pl.ds(nc, 16)], rel, mask=m)
            return nc + _scalar(plsc.all_reduce_population_count(m))

        nc = lax.fori_loop(0, _CH // 16, grp, jnp.int32(0))
        stag_s[pl.ds(nc, 16)] = zero16
        stag_d[pl.ds(nc, 16)] = sent16
        ncr = ((nc + 15) // 16) * 16
        pltpu.sync_copy(stag_s, srcc_hbm.at[pl.ds(pl.multiple_of(w * _ROWL + cnt16, 8), _CH + 16)])
        pltpu.sync_copy(stag_d, drelc_hbm.at[pl.ds(pl.multiple_of(w * _ROWL + cnt16, 8), _CH + 16)])
        return cnt16 + ncr

    cnt16 = lax.fori_loop(0, _NCHUNK, chunk_body, jnp.int32(0))
    pltpu.sync_copy(hist, indeg_hbm.at[pl.ds(pl.multiple_of(d0, 8), _SLAB)])
    cbuf[...] = jnp.full((16,), cnt16, jnp.int32)
    pltpu.sync_copy(cbuf, cnt_hbm.at[pl.ds(pl.multiple_of(w * 16, 8), 16)])


_prep_call = functools.partial(
    pl.kernel,
    out_type=(
        jax.ShapeDtypeStruct((_NPAD,), jnp.int32),
        jax.ShapeDtypeStruct((_NW * 16,), jnp.int32),
        jax.ShapeDtypeStruct((_NW * _ROWL,), jnp.int32),
        jax.ShapeDtypeStruct((_NW * _ROWL,), jnp.int32),
    ),
    mesh=_sc_mesh,
    compiler_params=pltpu.CompilerParams(needs_layout_passes=False),
    scratch_types=(
        pltpu.VMEM((_CH,), jnp.int32),
        pltpu.VMEM((_CH,), jnp.int32),
        pltpu.VMEM((_CH + 16,), jnp.int32),
        pltpu.VMEM((_CH + 16,), jnp.int32),
        pltpu.VMEM((_SLAB,), jnp.int32),
        pltpu.VMEM((16,), jnp.int32),
    ),
)(_prep_body)


def _sc_prep(dst, src):
    return _prep_call(dst, src)


_CG = 16
_ECH = 1024
_NGRP = _ECH // _CG
_ACCR = _SLAB + 8


def _normc_body(dinv, cnt, srcc, drelc, normc,
                idxbuf, drbuf, dsrcbuf, nbuf, dinvslab, cvec, sem):
    w = lax.axis_index("c") * 16 + lax.axis_index("s")
    d0 = w * _SLAB
    zf16 = jnp.zeros((16,), jnp.float32)
    pltpu.sync_copy(dinv.at[pl.ds(pl.multiple_of(d0, 8), _SLAB)],
                    dinvslab.at[pl.ds(0, _SLAB)])
    dinvslab[pl.ds(_SLAB, 16)] = zf16
    pltpu.sync_copy(cnt.at[pl.ds(pl.multiple_of(w * 16, 8), 16)], cvec)
    cw = jnp.max(cvec[...])
    nch = (cw + _ECH - 1) // _ECH

    def chunk(c, _):
        eoff = pl.multiple_of(w * _ROWL + c * _ECH, 8)
        pltpu.sync_copy(srcc.at[pl.ds(eoff, _ECH)], idxbuf)
        pltpu.sync_copy(drelc.at[pl.ds(eoff, _ECH)], drbuf)

        def fix(g, _):
            iv = idxbuf[pl.ds(pl.multiple_of(g * 16, 8), 16)]
            idxbuf[pl.ds(pl.multiple_of(g * 16, 8), 16)] = jnp.clip(iv, 0, N - 1)
            dv = drbuf[pl.ds(pl.multiple_of(g * 16, 8), 16)]
            drbuf[pl.ds(pl.multiple_of(g * 16, 8), 16)] = jnp.clip(dv, 0, _TRASH)
            return 0

        lax.fori_loop(0, _ECH // 16, fix, 0)
        pltpu.async_copy(dinv.at[idxbuf], dsrcbuf, sem).wait()

        def mk(g, _):
            dv = drbuf[pl.ds(pl.multiple_of(g * 16, 8), 16)]
            dd = plsc.load_gather(dinvslab, [dv])
            ds_ = dsrcbuf[pl.ds(pl.multiple_of(g * 16, 8), 16)]
            nbuf[pl.ds(pl.multiple_of(g * 16, 8), 16)] = ds_ * dd
            return 0

        lax.fori_loop(0, _ECH // 16, mk, 0)
        pltpu.sync_copy(nbuf, normc.at[pl.ds(eoff, _ECH)])
        return 0

    lax.fori_loop(0, nch, chunk, 0)


_normc_call = functools.partial(
    pl.kernel,
    out_type=jax.ShapeDtypeStruct((_NW * _ROWL,), jnp.float32),
    mesh=_sc_mesh,
    compiler_params=pltpu.CompilerParams(needs_layout_passes=False),
    scratch_types=(
        pltpu.VMEM((_ECH,), jnp.int32),
        pltpu.VMEM((_ECH,), jnp.int32),
        pltpu.VMEM((_ECH,), jnp.float32),
        pltpu.VMEM((_ECH,), jnp.float32),
        pltpu.VMEM((_SLAB + 16,), jnp.float32),
        pltpu.VMEM((16,), jnp.int32),
        pltpu.SemaphoreType.DMA,
    ),
)(_normc_body)


def _accum_group(acc, rowbuf, drv, nmv):
    for l in range(16):
        dr = drv[l]
        nm = None if nmv is None else nmv[l]

        def jq_body(jq, _):
            for ju in range(4):
                sl = pl.ds(pl.multiple_of(dr * 256 + jq * 64 + ju * 16, 8), 16)
                v = rowbuf[l, pl.ds(pl.multiple_of(jq * 64 + ju * 16, 8), 16)]
                if nm is not None:
                    v = v * nm
                acc[sl] = acc[sl] + v
            return 0

        lax.fori_loop(0, 4, jq_body, 0)


def _make_edge_pass(with_norm, with_epi=True):
    def body(*refs):
        if with_norm:
            (rows_hbm, dinv, b1, cnt, srcc, drelc, normc, out,
             acc, rbA, rbB, idxbuf, drbuf, nbuf, dinvslab, b1buf, cvec,
             semA, semB) = refs
        else:
            (rows_hbm, cnt, srcc, drelc, out,
             acc, rbA, rbB, idxbuf, drbuf, cvec, semA, semB) = refs
        w = lax.axis_index("c") * 16 + lax.axis_index("s")
        d0 = w * _SLAB
        zf16 = jnp.zeros((16,), jnp.float32)

        def zr(t, _):
            acc[pl.ds(pl.multiple_of(t * 16, 8), 16)] = zf16
            return 0

        lax.fori_loop(0, _ACCR * 16, zr, 0)
        if with_norm and with_epi:
            pltpu.sync_copy(dinv.at[pl.ds(pl.multiple_of(d0, 8), _SLAB)],
                            dinvslab.at[pl.ds(0, _SLAB)])
            dinvslab[pl.ds(_SLAB, 16)] = zf16
            pltpu.sync_copy(b1, b1buf)
        pltpu.sync_copy(cnt.at[pl.ds(pl.multiple_of(w * 16, 8), 16)], cvec)
        cw = jnp.max(cvec[...])
        nch = (cw + _ECH - 1) // _ECH

        def chunk(c, _):
            eoff = pl.multiple_of(w * _ROWL + c * _ECH, 8)
            pltpu.sync_copy(srcc.at[pl.ds(eoff, _ECH)], idxbuf)
            pltpu.sync_copy(drelc.at[pl.ds(eoff, _ECH)], drbuf)
            if with_norm:
                pltpu.sync_copy(normc.at[pl.ds(eoff, _ECH)], nbuf)

            iota16 = lax.iota(jnp.int32, 16)

            def fix(g, _):
                gsl = pl.ds(pl.multiple_of(g * 16, 8), 16)
                iv = idxbuf[gsl]
                idxbuf[gsl] = jnp.clip(iv, 0, N - 1)
                dv = drbuf[gsl]
                ev = (c * _ECH + g * 16 + iota16) < cw
                drbuf[gsl] = jnp.where(ev, jnp.clip(dv, 0, _TRASH), _TRASH)
                return 0

            lax.fori_loop(0, _ECH // 16, fix, 0)
            pltpu.async_copy(rows_hbm.at[idxbuf.at[pl.ds(0, 16)]], rbA, semA)
            pltpu.async_copy(rows_hbm.at[idxbuf.at[pl.ds(16, 16)]], rbB, semB)

            def pair(p, _):
                for par, rb, sem in ((0, rbA, semA), (1, rbB, semB)):
                    g = p * 2 + par
                    gsl = pl.ds(pl.multiple_of(g * 16, 8), 16)
                    pltpu.make_async_copy(rows_hbm.at[idxbuf.at[gsl]], rb, sem).wait()
                    drv = drbuf[gsl]
                    nmv = nbuf[gsl] if with_norm else None
                    _accum_group(acc, rb, drv, nmv)

                    @pl.when(p < _NGRP // 2 - 1)
                    def _():
                        g2 = g + 2
                        g2sl = pl.ds(pl.multiple_of(g2 * 16, 8), 16)
                        pltpu.async_copy(rows_hbm.at[idxbuf.at[g2sl]], rb, sem)

                return 0

            lax.fori_loop(0, _NGRP // 2, pair, 0)
            return 0

        lax.fori_loop(0, nch, chunk, 0)

        if with_norm and with_epi:
            def rowc(rc, _):
                pltpu.sync_copy(
                    rows_hbm.at[pl.ds(pl.multiple_of(d0 + rc * 16, 8), 16)], rbA)
                snv = dinvslab[pl.ds(pl.multiple_of(rc * 16, 8), 16)]
                for r in range(16):
                    sn = snv[r]
                    sn2 = sn * sn

                    def jq_body(jq, _):
                        for ju in range(4):
                            jo = jq * 64 + ju * 16
                            sl = pl.ds(pl.multiple_of(
                                (rc * 16 + r) * 256 + jo, 8), 16)
                            t = acc[sl] + rbA[r, pl.ds(pl.multiple_of(jo, 8), 16)] * sn2
                            t = t + b1buf[pl.ds(pl.multiple_of(jo, 8), 16)]
                            acc[sl] = jnp.maximum(t, 0.0)
                        return 0

                    lax.fori_loop(0, 4, jq_body, 0)
                return 0

            lax.fori_loop(0, _SLAB // 16, rowc, 0)
        pltpu.sync_copy(acc.at[pl.ds(0, _SLAB * 256)],
                        out.at[pl.ds(pl.multiple_of(d0 * 256, 8), _SLAB * 256)])

    return body


_conv1_call = functools.partial(
    pl.kernel,
    out_type=jax.ShapeDtypeStruct((_NPAD * 256,), jnp.float32),
    mesh=_sc_mesh,
    compiler_params=pltpu.CompilerParams(needs_layout_passes=False),
    scratch_types=(
        pltpu.VMEM((_ACCR * 256,), jnp.float32),
        pltpu.VMEM((_CG, 256), jnp.float32),
        pltpu.VMEM((_CG, 256), jnp.float32),
        pltpu.VMEM((_ECH,), jnp.int32),
        pltpu.VMEM((_ECH,), jnp.int32),
        pltpu.VMEM((_ECH,), jnp.float32),
        pltpu.VMEM((_SLAB + 16,), jnp.float32),
        pltpu.VMEM((256,), jnp.float32),
        pltpu.VMEM((16,), jnp.int32),
        pltpu.SemaphoreType.DMA,
        pltpu.SemaphoreType.DMA,
    ),
)(_make_edge_pass(True, True))

_agg_call = functools.partial(
    pl.kernel,
    out_type=jax.ShapeDtypeStruct((_NPAD * 256,), jnp.float32),
    mesh=_sc_mesh,
    compiler_params=pltpu.CompilerParams(needs_layout_passes=False),
    scratch_types=(
        pltpu.VMEM((_ACCR * 256,), jnp.float32),
        pltpu.VMEM((_CG, 256), jnp.float32),
        pltpu.VMEM((_CG, 256), jnp.float32),
        pltpu.VMEM((_ECH,), jnp.int32),
        pltpu.VMEM((_ECH,), jnp.int32),
        pltpu.VMEM((_ECH,), jnp.float32),
        pltpu.VMEM((_SLAB + 16,), jnp.float32),
        pltpu.VMEM((256,), jnp.float32),
        pltpu.VMEM((16,), jnp.int32),
        pltpu.SemaphoreType.DMA,
        pltpu.SemaphoreType.DMA,
    ),
)(_make_edge_pass(True, False))


def _dinv_body(i_ref, o_ref):
    o_ref[...] = 1.0 / jnp.sqrt(i_ref[...].astype(jnp.float32) + 1.0)


def _pallas_dinv(indeg):
    r = pl.pallas_call(
        _dinv_body,
        out_shape=jax.ShapeDtypeStruct((_NPAD // 128, 128), jnp.float32),
    )(indeg.reshape(_NPAD // 128, 128))
    return r.reshape(-1)


_SR, _SC = 128, 128


def _sort_body(s_ref, i_ref, os_ref, oi_ref):
    s = s_ref[...]
    ix = i_ref[...]
    row = jax.lax.broadcasted_iota(jnp.int32, (_SR, _SC), 0)
    col = jax.lax.broadcasted_iota(jnp.int32, (_SR, _SC), 1)
    n = _SR * _SC
    k = 2
    while k <= n:
        j = k // 2
        while j >= 1:
            if j < _SC:
                bit_j = (col & j) != 0
                ps = jnp.where(bit_j, jnp.roll(s, j, 1), jnp.roll(s, -j, 1))
                pi = jnp.where(bit_j, jnp.roll(ix, j, 1), jnp.roll(ix, -j, 1))
            else:
                jr = j // _SC
                bit_j = (row & jr) != 0
                ps = jnp.where(bit_j, jnp.roll(s, jr, 0), jnp.roll(s, -jr, 0))
                pi = jnp.where(bit_j, jnp.roll(ix, jr, 0), jnp.roll(ix, -jr, 0))
            if k < _SC:
                bit_k = (col & k) != 0
            elif k < n:
                bit_k = (row & (k // _SC)) != 0
            else:
                bit_k = jnp.zeros((_SR, _SC), dtype=bool)
            a_first = (s > ps) | ((s == ps) & (ix < pi))
            keep_early = bit_k == bit_j
            es = jnp.where(a_first, s, ps)
            ei = jnp.where(a_first, ix, pi)
            ls = jnp.where(a_first, ps, s)
            li = jnp.where(a_first, pi, ix)
            s = jnp.where(keep_early, es, ls)
            ix = jnp.where(keep_early, ei, li)
            j //= 2
        k *= 2
    os_ref[...] = s
    oi_ref[...] = ix


def _pallas_sort(score_pad, idx_pad):
    return pl.pallas_call(
        _sort_body,
        out_shape=(
            jax.ShapeDtypeStruct((_SR, _SC), jnp.float32),
            jax.ShapeDtypeStruct((_SR, _SC), jnp.int32),
        ),
    )(score_pad, idx_pad)


def _topk_sorted(score):
    pad = jnp.full((_SR * _SC - N,), -jnp.inf, jnp.float32)
    sp = jnp.concatenate([score, pad]).reshape(_SR, _SC)
    ip = jnp.arange(_SR * _SC, dtype=jnp.int32).reshape(_SR, _SC)
    ss, si = _pallas_sort(sp, ip)
    return ss.reshape(-1)[:K], si.reshape(-1)[:K]


def _gcn_conv(x, src, dst, ew, W, b, n):
    loop = jnp.arange(n, dtype=src.dtype)
    s = jnp.concatenate([src, loop])
    d = jnp.concatenate([dst, loop])
    w = jnp.concatenate([ew, jnp.ones((n,), dtype=x.dtype)])
    deg = jnp.zeros((n,), dtype=x.dtype).at[d].add(w)
    dinv = jnp.where(deg > 0, 1.0 / jnp.sqrt(jnp.where(deg > 0, deg, 1.0)), 0.0)
    norm = dinv[s] * dinv[d] * w
    h = x @ W
    out = jnp.zeros((n, W.shape[1]), dtype=x.dtype).at[d].add(h[s] * norm[:, None])
    return out + b


def _graph_conv(x, src, dst, Wrel, Wroot, b, n):
    agg = jnp.zeros((n, x.shape[1]), dtype=x.dtype).at[dst].add(x[src])
    return agg @ Wrel + x @ Wroot + b


def _gcn_conv1_ordered(x, src, dst, W, b, n, order, indeg, base, deg):
    dinv = 1.0 / jnp.sqrt(deg)
    h = _pallas_mm(x, W)
    src_s = src[order]
    dst_s = dst[order]
    norm_s = dinv[src_s] * dinv[dst_s] * 1.0
    vals_s = h[src_s] * norm_s[:, None]
    acc = _ordered_scatter_rows(vals_s, base, indeg, n, W.shape[1])
    self_norm = dinv * dinv * 1.0
    acc = acc + h * self_norm[:, None]
    return acc + b


def _graph_conv_ordered(x, src, Wrel, Wroot, b, n, order, indeg, base):
    vals_s = x[src[order]]
    agg = _ordered_scatter_rows(vals_s, base, indeg, n, x.shape[1])
    return _pallas_score(agg, x, Wrel, Wroot, b, bm=1000)


def kernel(x, edge_index, W1, b1, W2, b2, Wg_rel, Wg_root, bg):
    src = edge_index[0]
    dst = edge_index[1]
    ew = jnp.ones((E,), dtype=x.dtype)
    src32 = src.astype(jnp.int32)
    dst32 = dst.astype(jnp.int32)
    indeg_sc, cnts_sc, srcc_sc, drelc_sc = _sc_prep(dst32, src32)
    hpre = _pallas_mm(x, W1)
    hpre_pad = jnp.pad(hpre, ((0, _NPAD - N), (0, 0)))
    dinv1 = 1.0 / jnp.sqrt(indeg_sc.astype(jnp.float32) + 1.0)
    normc = _normc_call(dinv1, cnts_sc, srcc_sc, drelc_sc)
    h_pad = _conv1_call(hpre_pad, dinv1, b1, cnts_sc, srcc_sc, drelc_sc, normc).reshape(_NPAD, 256)
    h = h_pad[:N]
    ones_dinv = jnp.ones((_NPAD,), jnp.float32)
    normc1 = _normc_call(ones_dinv, cnts_sc, srcc_sc, drelc_sc)
    agg = _agg_call(h_pad, dinv1, b1, cnts_sc, srcc_sc, drelc_sc,
                    normc1).reshape(_NPAD, 256)[:N]
    score = _pallas_score(agg, h, Wg_rel, Wg_root, bg).reshape(-1)
    vals, perm = _topk_sorted(score)
    xp = h[perm] * vals[:, None]
    keep = jnp.zeros((N,), dtype=bool).at[perm].set(True)
    new_idx = jnp.zeros((N,), dtype=jnp.int32).at[perm].set(jnp.arange(K, dtype=jnp.int32))
    emask = keep[src] & keep[dst]
    s2 = jnp.where(emask, new_idx[src], 0)
    d2 = jnp.where(emask, new_idx[dst], 0)
    ew2 = emask.astype(x.dtype)
    out = jax.nn.relu(_gcn_conv(xp, s2, d2, ew2, W2, b2, K))
    return out, jnp.stack([s2, d2]), emask
